# baseline (speedup 1.0000x reference)
import functools

import jax
import jax.numpy as jnp
from jax import lax
from jax.experimental import pallas as pl
from jax.experimental.pallas import tpu as pltpu
from jax.experimental.pallas import tpu_sc as plsc

MAX_TOKEN = 512
N_NODES = 705
B = 32
H = 768
NB = B * N_NODES
HV = H // 16
NC = 2
NS = 16
NW = NC * NS
RPW = NB // NW
C = 32
TCHED = 736
KEY_STRIDE = 32768


def _init_body(hid_ref, w_ref, mask_ref, out_ref):
    out_ref[0, :MAX_TOKEN, :] = hid_ref[0]
    m = (mask_ref[0, 0, MAX_TOKEN:] == 1).astype(jnp.float32)
    out_ref[0, MAX_TOKEN:, :] = w_ref[...] * m[:, None]


def _build_flat(hidden_states, W, st_mask):
    return pl.pallas_call(
        _init_body,
        grid=(B,),
        in_specs=[
            pl.BlockSpec((1, MAX_TOKEN, H), lambda b: (b, 0, 0)),
            pl.BlockSpec((N_NODES - MAX_TOKEN, H), lambda b: (0, 0)),
            pl.BlockSpec((1, 1, N_NODES), lambda b: (b, 0, 0)),
        ],
        out_specs=pl.BlockSpec((1, N_NODES, H), lambda b: (b, 0, 0)),
        out_shape=jax.ShapeDtypeStruct((B, N_NODES, H), jnp.float32),
    )(hidden_states, W, st_mask.reshape(B, 1, N_NODES))


@functools.lru_cache(maxsize=None)
def _mesh():
    return plsc.VectorSubcoreMesh(
        core_axis_name="c", subcore_axis_name="s", num_cores=NC, num_subcores=NS
    )


def _wid():
    return lax.axis_index("s") * NC + lax.axis_index("c")


def _phase_a_body(EP,
                  flat_hbm, src_hbm, tgt_hbm, starts_hbm,
                  delta_hbm, touched_hbm, counts_hbm,
                  src_v, tgt_v, starts_v, idx_v, rows_v, acc_v, avg_v,
                  touched_v, cnt_v, sem):
    w = _wid()
    pltpu.sync_copy(src_hbm, src_v)
    pltpu.sync_copy(tgt_hbm, tgt_v)
    pltpu.sync_copy(starts_hbm, starts_v)
    sv = starts_v[pl.ds(w, 16)]
    lo = sv[0]
    hi = sv[1]
    c0 = lo // C
    c1 = (hi + C - 1) // C

    def chunk_body(cc, carry):
        base = cc * C
        for j in range(C // 16):
            idx_v[pl.ds(j * 16, 16)] = src_v[pl.ds(base + j * 16, 16)]
        pltpu.async_copy(flat_hbm.at[idx_v], rows_v, sem).wait()

        e_begin = jnp.maximum(lo, base)
        e_end = jnp.minimum(hi, base + C)

        def edge_body(e, car):
            cur, cnt, nw = car
            t = tgt_v[pl.ds(e, 16)][0]
            slot = e - base
            is_new = t != cur
            do_flush = jnp.logical_and(is_new, cnt > 0)

            @pl.when(do_flush)
            def _():
                cntv = jnp.full((16,), cnt, dtype=jnp.int32).astype(jnp.float32)
                invv = 1.0 / cntv
                for j in range(HV):
                    avg_v[pl.ds(j * 16, 16)] = acc_v[pl.ds(j * 16, 16)] * invv
                pltpu.sync_copy(avg_v, delta_hbm.at[cur])
                touched_v[pl.ds(nw, 16)] = jnp.full((16,), cur, dtype=jnp.int32)

            nw = jnp.where(do_flush, nw + 1, nw)

            @pl.when(is_new)
            def _():
                for j in range(HV):
                    acc_v[pl.ds(j * 16, 16)] = rows_v[slot, pl.ds(j * 16, 16)]

            @pl.when(jnp.logical_not(is_new))
            def _():
                for j in range(HV):
                    acc_v[pl.ds(j * 16, 16)] = (
                        acc_v[pl.ds(j * 16, 16)] + rows_v[slot, pl.ds(j * 16, 16)]
                    )

            cnt = jnp.where(is_new, 1, cnt + 1)
            return t, cnt, nw

        return lax.fori_loop(e_begin, e_end, edge_body, carry)

    cur, cnt, nw = lax.fori_loop(
        c0, c1, chunk_body,
        (jnp.int32(-1), jnp.int32(0), jnp.int32(0)),
    )

    @pl.when(cnt > 0)
    def _():
        cntv = jnp.full((16,), cnt, dtype=jnp.int32).astype(jnp.float32)
        invv = 1.0 / cntv
        for j in range(HV):
            avg_v[pl.ds(j * 16, 16)] = acc_v[pl.ds(j * 16, 16)] * invv
        pltpu.sync_copy(avg_v, delta_hbm.at[cur])
        touched_v[pl.ds(nw, 16)] = jnp.full((16,), cur, dtype=jnp.int32)

    nw = jnp.where(cnt > 0, nw + 1, nw)

    pltpu.sync_copy(touched_v, touched_hbm.at[w])
    cnt_v[...] = jnp.full((16,), nw, dtype=jnp.int32)
    pltpu.sync_copy(cnt_v, counts_hbm.at[w])


@functools.lru_cache(maxsize=None)
def _make_phase_a(EP):
    return pl.kernel(
        functools.partial(_phase_a_body, EP),
        out_type=(
            jax.ShapeDtypeStruct((NB, H), jnp.float32),
            jax.ShapeDtypeStruct((NW, TCHED), jnp.int32),
            jax.ShapeDtypeStruct((NW, 16), jnp.int32),
        ),
        mesh=_mesh(),
        scratch_types=[
            pltpu.VMEM((EP + 16,), jnp.int32),
            pltpu.VMEM((EP + 16,), jnp.int32),
            pltpu.VMEM((48,), jnp.int32),
            pltpu.VMEM((C,), jnp.int32),
            pltpu.VMEM((C, H), jnp.float32),
            pltpu.VMEM((H,), jnp.float32),
            pltpu.VMEM((H,), jnp.float32),
            pltpu.VMEM((TCHED,), jnp.int32),
            pltpu.VMEM((16,), jnp.int32),
            pltpu.SemaphoreType.DMA,
        ],
    )


def _phase_b_body(delta_hbm, touched_hbm, counts_hbm, flat_hbm,
                  touched_v, idx_v, fa_v, fb_v, cnt_v, sem):
    w = _wid()
    pltpu.sync_copy(touched_hbm.at[w], touched_v)
    pltpu.sync_copy(counts_hbm.at[w], cnt_v)
    nw = cnt_v[...][0]
    nchunks = (nw + 15) // 16

    def body(c, _):
        idx_v[...] = touched_v[pl.ds(c * 16, 16)]
        pltpu.async_copy(flat_hbm.at[idx_v], fa_v, sem).wait()
        pltpu.async_copy(delta_hbm.at[idx_v], fb_v, sem).wait()

        def row_body(r, _):
            for j in range(HV):
                fa_v[r, pl.ds(j * 16, 16)] = (
                    fa_v[r, pl.ds(j * 16, 16)] + fb_v[r, pl.ds(j * 16, 16)]
                )
            return 0

        lax.fori_loop(0, 16, row_body, 0)
        pltpu.async_copy(fa_v, flat_hbm.at[idx_v], sem).wait()
        return 0

    lax.fori_loop(0, nchunks, body, 0)


def _copy_block(src_hbm, dst_hbm, offs_v):
    w = _wid()
    ov = offs_v[pl.ds(w, 16)]
    off = pl.multiple_of(ov[0], 8)
    blk = ov[1] - ov[0]
    pltpu.sync_copy(src_hbm.at[pl.ds(off, 704)], dst_hbm.at[pl.ds(off, 704)])

    @pl.when(blk > 704)
    def _():
        off2 = pl.multiple_of(off + 704, 8)
        pltpu.sync_copy(src_hbm.at[pl.ds(off2, 8)], dst_hbm.at[pl.ds(off2, 8)])


def _phase_b_final_body(delta_hbm, touched_hbm, counts_hbm, offs_hbm, flat_hbm,
                        out_hbm, touched_v, idx_v, fa_v, fb_v, cnt_v, offs_v,
                        sem):
    _phase_b_body(delta_hbm, touched_hbm, counts_hbm, flat_hbm,
                  touched_v, idx_v, fa_v, fb_v, cnt_v, sem)
    pltpu.sync_copy(offs_hbm, offs_v)
    _copy_block(flat_hbm, out_hbm, offs_v)


_PHASE_B_SCRATCH = [
    pltpu.VMEM((TCHED,), jnp.int32),
    pltpu.VMEM((16,), jnp.int32),
    pltpu.VMEM((16, H), jnp.float32),
    pltpu.VMEM((16, H), jnp.float32),
    pltpu.VMEM((16,), jnp.int32),
    pltpu.SemaphoreType.DMA,
]


@functools.lru_cache(maxsize=None)
def _make_phase_b():
    return pl.kernel(
        _phase_b_body,
        out_type=(),
        mesh=_mesh(),
        scratch_types=list(_PHASE_B_SCRATCH),
    )


@functools.lru_cache(maxsize=None)
def _make_phase_b_final():
    return pl.kernel(
        _phase_b_final_body,
        out_type=jax.ShapeDtypeStruct((NB, H), jnp.float32),
        mesh=_mesh(),
        scratch_types=list(_PHASE_B_SCRATCH[:-1])
        + [pltpu.VMEM((48,), jnp.int32), pltpu.SemaphoreType.DMA],
    )


def kernel(hidden_states, W, st_mask, edges_src, edges_tgt, edges_type, edges_pos):
    E = edges_src.shape[0]
    EP = ((E + C - 1) // C) * C

    bucket = jnp.where(
        edges_type == 1, 0,
        jnp.where(edges_type == 6, 1, jnp.where(edges_type == 11, 2, 3)),
    ).astype(jnp.int32)
    key = bucket * KEY_STRIDE + edges_tgt
    _, src_s, tgt_s = lax.sort((key, edges_src, edges_tgt), num_keys=1)
    src_s = jnp.pad(src_s, (0, EP + 16 - E))
    tgt_s = jnp.pad(tgt_s, (0, EP + 16 - E))

    offs = ((jnp.arange(33, dtype=jnp.int32) * RPW) // 8 * 8).astype(jnp.int32)
    offs_p = jnp.pad(offs, (0, 15))

    bnd = (jnp.arange(3, dtype=jnp.int32)[:, None] * KEY_STRIDE
           + offs[None, :])
    starts = jnp.sum(key[None, :] < bnd.reshape(-1)[:, None], axis=1,
                     dtype=jnp.int32)
    starts = starts.reshape(3, 33)
    starts = jnp.pad(starts, ((0, 0), (0, 15)))

    flat0 = _build_flat(hidden_states, W, st_mask).reshape(NB, H)
    flat_ref = jax.new_ref(flat0)

    phase_a = _make_phase_a(EP)
    phase_b = _make_phase_b()
    delta, touched, counts = phase_a(flat0, src_s, tgt_s, starts[0])
    phase_b(delta, touched, counts, flat_ref)
    for k in range(1, 2):
        delta, touched, counts = phase_a(flat_ref, src_s, tgt_s, starts[k])
        phase_b(delta, touched, counts, flat_ref)
    delta, touched, counts = phase_a(flat_ref, src_s, tgt_s, starts[2])
    out = _make_phase_b_final()(delta, touched, counts, offs_p, flat_ref)

    return out.reshape(B, N_NODES, H)


# speedup vs baseline: 5.9027x; 5.9027x over previous
import functools

import jax
import jax.numpy as jnp
from jax import lax
from jax.experimental import pallas as pl
from jax.experimental.pallas import tpu as pltpu
from jax.experimental.pallas import tpu_sc as plsc

MAX_TOKEN = 512
N_NODES = 705
B = 32
H = 768
NB = B * N_NODES
HV = H // 16
NC = 2
NS = 16
NW = NC * NS
RPW = NB // NW
C = 32
TCHED = 736
KEY_STRIDE = 32768


def _init_body(hid_ref, w_ref, mask_ref, out_ref):
    out_ref[0, :MAX_TOKEN, :] = hid_ref[0]
    m = (mask_ref[0, 0, MAX_TOKEN:] == 1).astype(jnp.float32)
    out_ref[0, MAX_TOKEN:, :] = w_ref[...] * m[:, None]


def _build_flat(hidden_states, W, st_mask):
    return pl.pallas_call(
        _init_body,
        grid=(B,),
        in_specs=[
            pl.BlockSpec((1, MAX_TOKEN, H), lambda b: (b, 0, 0)),
            pl.BlockSpec((N_NODES - MAX_TOKEN, H), lambda b: (0, 0)),
            pl.BlockSpec((1, 1, N_NODES), lambda b: (b, 0, 0)),
        ],
        out_specs=pl.BlockSpec((1, N_NODES, H), lambda b: (b, 0, 0)),
        out_shape=jax.ShapeDtypeStruct((B, N_NODES, H), jnp.float32),
    )(hidden_states, W, st_mask.reshape(B, 1, N_NODES))


@functools.lru_cache(maxsize=None)
def _mesh():
    return plsc.VectorSubcoreMesh(
        core_axis_name="c", subcore_axis_name="s", num_cores=NC, num_subcores=NS
    )


def _wid():
    return lax.axis_index("s") * NC + lax.axis_index("c")


def _phase_a_body(EP,
                  flat_hbm, src_hbm, tgt_hbm, starts_hbm,
                  delta_hbm, touched_hbm, counts_hbm,
                  src_v, tgt_v, starts_v, idx_v, rows_v, acc_v, avg_v,
                  touched_v, cnt_v, sem):
    w = _wid()
    pltpu.sync_copy(src_hbm, src_v)
    pltpu.sync_copy(tgt_hbm, tgt_v)
    pltpu.sync_copy(starts_hbm, starts_v)
    sv = starts_v[pl.ds(w, 16)]
    lo = sv[0]
    hi = sv[1]
    c0 = lo // C
    c1 = (hi + C - 1) // C

    def chunk_body(cc, carry):
        base = cc * C
        for j in range(C // 16):
            idx_v[pl.ds(j * 16, 16)] = src_v[pl.ds(base + j * 16, 16)]
        pltpu.async_copy(flat_hbm.at[idx_v], rows_v, sem).wait()

        e_begin = jnp.maximum(lo, base)
        e_end = jnp.minimum(hi, base + C)

        def edge_body(e, car):
            cur, cnt, nw = car
            t = tgt_v[pl.ds(e, 16)][0]
            slot = e - base
            is_new = t != cur
            do_flush = jnp.logical_and(is_new, cnt > 0)

            @pl.when(do_flush)
            def _():
                cntv = jnp.full((16,), cnt, dtype=jnp.int32).astype(jnp.float32)
                invv = 1.0 / cntv
                for j in range(HV):
                    avg_v[pl.ds(j * 16, 16)] = acc_v[pl.ds(j * 16, 16)] * invv
                pltpu.sync_copy(avg_v, delta_hbm.at[cur])
                touched_v[pl.ds(nw, 16)] = jnp.full((16,), cur, dtype=jnp.int32)

            nw = jnp.where(do_flush, nw + 1, nw)

            @pl.when(is_new)
            def _():
                for j in range(HV):
                    acc_v[pl.ds(j * 16, 16)] = rows_v[slot, pl.ds(j * 16, 16)]

            @pl.when(jnp.logical_not(is_new))
            def _():
                for j in range(HV):
                    acc_v[pl.ds(j * 16, 16)] = (
                        acc_v[pl.ds(j * 16, 16)] + rows_v[slot, pl.ds(j * 16, 16)]
                    )

            cnt = jnp.where(is_new, 1, cnt + 1)
            return t, cnt, nw

        return lax.fori_loop(e_begin, e_end, edge_body, carry)

    cur, cnt, nw = lax.fori_loop(
        c0, c1, chunk_body,
        (jnp.int32(-1), jnp.int32(0), jnp.int32(0)),
    )

    @pl.when(cnt > 0)
    def _():
        cntv = jnp.full((16,), cnt, dtype=jnp.int32).astype(jnp.float32)
        invv = 1.0 / cntv
        for j in range(HV):
            avg_v[pl.ds(j * 16, 16)] = acc_v[pl.ds(j * 16, 16)] * invv
        pltpu.sync_copy(avg_v, delta_hbm.at[cur])
        touched_v[pl.ds(nw, 16)] = jnp.full((16,), cur, dtype=jnp.int32)

    nw = jnp.where(cnt > 0, nw + 1, nw)

    pltpu.sync_copy(touched_v, touched_hbm.at[w])
    cnt_v[...] = jnp.full((16,), nw, dtype=jnp.int32)
    pltpu.sync_copy(cnt_v, counts_hbm.at[w])


@functools.lru_cache(maxsize=None)
def _make_phase_a(EP):
    return pl.kernel(
        functools.partial(_phase_a_body, EP),
        out_type=(
            jax.ShapeDtypeStruct((NB, H), jnp.float32),
            jax.ShapeDtypeStruct((NW, TCHED), jnp.int32),
            jax.ShapeDtypeStruct((NW, 16), jnp.int32),
        ),
        mesh=_mesh(),
        scratch_types=[
            pltpu.VMEM((EP + 16,), jnp.int32),
            pltpu.VMEM((EP + 16,), jnp.int32),
            pltpu.VMEM((48,), jnp.int32),
            pltpu.VMEM((C,), jnp.int32),
            pltpu.VMEM((C, H), jnp.float32),
            pltpu.VMEM((H,), jnp.float32),
            pltpu.VMEM((H,), jnp.float32),
            pltpu.VMEM((TCHED,), jnp.int32),
            pltpu.VMEM((16,), jnp.int32),
            pltpu.SemaphoreType.DMA,
        ],
    )


def _phase_b_body(delta_hbm, touched_hbm, counts_hbm, flat_hbm,
                  touched_v, idx_v, fa_v, fb_v, cnt_v, sem):
    w = _wid()
    pltpu.sync_copy(touched_hbm.at[w], touched_v)
    pltpu.sync_copy(counts_hbm.at[w], cnt_v)
    nw = cnt_v[...][0]
    nchunks = (nw + 15) // 16

    def body(c, _):
        idx_v[...] = touched_v[pl.ds(c * 16, 16)]
        pltpu.async_copy(flat_hbm.at[idx_v], fa_v, sem).wait()
        pltpu.async_copy(delta_hbm.at[idx_v], fb_v, sem).wait()

        def row_body(r, _):
            for j in range(HV):
                fa_v[r, pl.ds(j * 16, 16)] = (
                    fa_v[r, pl.ds(j * 16, 16)] + fb_v[r, pl.ds(j * 16, 16)]
                )
            return 0

        lax.fori_loop(0, 16, row_body, 0)
        pltpu.async_copy(fa_v, flat_hbm.at[idx_v], sem).wait()
        return 0

    lax.fori_loop(0, nchunks, body, 0)


_PHASE_B_SCRATCH = [
    pltpu.VMEM((TCHED,), jnp.int32),
    pltpu.VMEM((16,), jnp.int32),
    pltpu.VMEM((16, H), jnp.float32),
    pltpu.VMEM((16, H), jnp.float32),
    pltpu.VMEM((16,), jnp.int32),
    pltpu.SemaphoreType.DMA,
]


@functools.lru_cache(maxsize=None)
def _make_phase_b():
    return pl.kernel(
        _phase_b_body,
        out_type=(),
        mesh=_mesh(),
        scratch_types=list(_PHASE_B_SCRATCH),
    )


def kernel(hidden_states, W, st_mask, edges_src, edges_tgt, edges_type, edges_pos):
    E = edges_src.shape[0]
    EP = ((E + C - 1) // C) * C

    bucket = jnp.where(
        edges_type == 1, 0,
        jnp.where(edges_type == 6, 1, jnp.where(edges_type == 11, 2, 3)),
    ).astype(jnp.int32)
    key = bucket * KEY_STRIDE + edges_tgt
    _, src_s, tgt_s = lax.sort((key, edges_src, edges_tgt), num_keys=1)
    src_s = jnp.pad(src_s, (0, EP + 16 - E))
    tgt_s = jnp.pad(tgt_s, (0, EP + 16 - E))

    offs = ((jnp.arange(33, dtype=jnp.int32) * RPW) // 8 * 8).astype(jnp.int32)

    bnd = (jnp.arange(3, dtype=jnp.int32)[:, None] * KEY_STRIDE
           + offs[None, :])
    starts = jnp.sum(key[None, :] < bnd.reshape(-1)[:, None], axis=1,
                     dtype=jnp.int32)
    starts = starts.reshape(3, 33)
    starts = jnp.pad(starts, ((0, 0), (0, 15)))

    flat0 = _build_flat(hidden_states, W, st_mask).reshape(NB, H)
    flat_ref = jax.new_ref(flat0)

    phase_a = _make_phase_a(EP)
    phase_b = _make_phase_b()
    for k in range(3):
        delta, touched, counts = phase_a(flat_ref, src_s, tgt_s, starts[k])
        phase_b(delta, touched, counts, flat_ref)

    return flat_ref[...].reshape(B, N_NODES, H)


# speedup vs baseline: 6.2229x; 1.0542x over previous
import functools

import jax
import jax.numpy as jnp
from jax import lax
from jax.experimental import pallas as pl
from jax.experimental.pallas import tpu as pltpu
from jax.experimental.pallas import tpu_sc as plsc

MAX_TOKEN = 512
N_NODES = 705
B = 32
H = 768
NB = B * N_NODES
HV = H // 16
NC = 2
NS = 16
NW = NC * NS
RPW = NB // NW
C = 32
G = 128
TCHED = 736
KEY_STRIDE = 32768


def _init_body(hid_ref, w_ref, mask_ref, out_ref):
    out_ref[0, :MAX_TOKEN, :] = hid_ref[0]
    m = (mask_ref[0, 0, MAX_TOKEN:] == 1).astype(jnp.float32)
    out_ref[0, MAX_TOKEN:, :] = w_ref[...] * m[:, None]


def _build_flat(hidden_states, W, st_mask):
    return pl.pallas_call(
        _init_body,
        grid=(B,),
        in_specs=[
            pl.BlockSpec((1, MAX_TOKEN, H), lambda b: (b, 0, 0)),
            pl.BlockSpec((N_NODES - MAX_TOKEN, H), lambda b: (0, 0)),
            pl.BlockSpec((1, 1, N_NODES), lambda b: (b, 0, 0)),
        ],
        out_specs=pl.BlockSpec((1, N_NODES, H), lambda b: (b, 0, 0)),
        out_shape=jax.ShapeDtypeStruct((B, N_NODES, H), jnp.float32),
    )(hidden_states, W, st_mask.reshape(B, 1, N_NODES))


@functools.lru_cache(maxsize=None)
def _mesh():
    return plsc.VectorSubcoreMesh(
        core_axis_name="c", subcore_axis_name="s", num_cores=NC, num_subcores=NS
    )


def _wid():
    return lax.axis_index("s") * NC + lax.axis_index("c")


def _phase_a_body(EP,
                  flat_hbm, src_hbm, tgt_hbm, starts_hbm,
                  delta_hbm, touched_hbm, counts_hbm,
                  srcg_v, tgtg_v, starts_v, idx_v, rows_v, acc_v, avg_v,
                  touched_v, cnt_v, sem):
    w = _wid()
    pltpu.sync_copy(starts_hbm, starts_v)
    sv = starts_v[pl.ds(w, 16)]
    lo = sv[0]
    hi = sv[1]
    g0 = lo // G
    g1 = (hi + G - 1) // G
    c0 = lo // C
    c1 = (hi + C - 1) // C
    GC = G // C

    def group_body(gg, carry):
        gb = pl.multiple_of(gg * G, G)
        pltpu.sync_copy(src_hbm.at[pl.ds(gb, G)], srcg_v)
        pltpu.sync_copy(tgt_hbm.at[pl.ds(gb, 2 * G)], tgtg_v)

        def chunk_body(cc, carry):
            base = cc * C
            off = base - gb
            for j in range(C // 16):
                idx_v[pl.ds(j * 16, 16)] = srcg_v[pl.ds(off + j * 16, 16)]
            pltpu.async_copy(flat_hbm.at[idx_v], rows_v, sem).wait()

            e_begin = jnp.maximum(lo, base)
            e_end = jnp.minimum(hi, base + C)

            def edge_body(e, car):
                cur, cnt, nw = car
                t = tgtg_v[pl.ds(e - gb, 16)][0]
                slot = e - base
                is_new = t != cur
                do_flush = jnp.logical_and(is_new, cnt > 0)

                @pl.when(do_flush)
                def _():
                    cntv = jnp.full((16,), cnt, dtype=jnp.int32).astype(jnp.float32)
                    invv = 1.0 / cntv
                    for j in range(HV):
                        avg_v[pl.ds(j * 16, 16)] = acc_v[pl.ds(j * 16, 16)] * invv
                    pltpu.sync_copy(avg_v, delta_hbm.at[cur])
                    touched_v[pl.ds(nw, 16)] = jnp.full((16,), cur, dtype=jnp.int32)

                nw = jnp.where(do_flush, nw + 1, nw)

                @pl.when(is_new)
                def _():
                    for j in range(HV):
                        acc_v[pl.ds(j * 16, 16)] = rows_v[slot, pl.ds(j * 16, 16)]

                @pl.when(jnp.logical_not(is_new))
                def _():
                    for j in range(HV):
                        acc_v[pl.ds(j * 16, 16)] = (
                            acc_v[pl.ds(j * 16, 16)] + rows_v[slot, pl.ds(j * 16, 16)]
                        )

                cnt = jnp.where(is_new, 1, cnt + 1)
                return t, cnt, nw

            return lax.fori_loop(e_begin, e_end, edge_body, carry)

        cc_lo = jnp.maximum(c0, gg * GC)
        cc_hi = jnp.minimum(c1, (gg + 1) * GC)
        return lax.fori_loop(cc_lo, cc_hi, chunk_body, carry)

    cur, cnt, nw = lax.fori_loop(
        g0, g1, group_body,
        (jnp.int32(-1), jnp.int32(0), jnp.int32(0)),
    )

    @pl.when(cnt > 0)
    def _():
        cntv = jnp.full((16,), cnt, dtype=jnp.int32).astype(jnp.float32)
        invv = 1.0 / cntv
        for j in range(HV):
            avg_v[pl.ds(j * 16, 16)] = acc_v[pl.ds(j * 16, 16)] * invv
        pltpu.sync_copy(avg_v, delta_hbm.at[cur])
        touched_v[pl.ds(nw, 16)] = jnp.full((16,), cur, dtype=jnp.int32)

    nw = jnp.where(cnt > 0, nw + 1, nw)

    pltpu.sync_copy(touched_v, touched_hbm.at[w])
    cnt_v[...] = jnp.full((16,), nw, dtype=jnp.int32)
    pltpu.sync_copy(cnt_v, counts_hbm.at[w])


@functools.lru_cache(maxsize=None)
def _make_phase_a(EP):
    return pl.kernel(
        functools.partial(_phase_a_body, EP),
        out_type=(
            jax.ShapeDtypeStruct((NB, H), jnp.float32),
            jax.ShapeDtypeStruct((NW, TCHED), jnp.int32),
            jax.ShapeDtypeStruct((NW, 16), jnp.int32),
        ),
        mesh=_mesh(),
        scratch_types=[
            pltpu.VMEM((G,), jnp.int32),
            pltpu.VMEM((2 * G,), jnp.int32),
            pltpu.VMEM((48,), jnp.int32),
            pltpu.VMEM((C,), jnp.int32),
            pltpu.VMEM((C, H), jnp.float32),
            pltpu.VMEM((H,), jnp.float32),
            pltpu.VMEM((H,), jnp.float32),
            pltpu.VMEM((TCHED,), jnp.int32),
            pltpu.VMEM((16,), jnp.int32),
            pltpu.SemaphoreType.DMA,
        ],
    )


def _phase_b_body(delta_hbm, touched_hbm, counts_hbm, flat_hbm,
                  touched_v, idx_v, fa_v, fb_v, cnt_v, sem, sem2):
    w = _wid()
    pltpu.sync_copy(touched_hbm.at[w], touched_v)
    pltpu.sync_copy(counts_hbm.at[w], cnt_v)
    nw = cnt_v[...][0]
    nchunks = (nw + 15) // 16

    def body(c, _):
        idx_v[...] = touched_v[pl.ds(c * 16, 16)]
        cpa = pltpu.async_copy(flat_hbm.at[idx_v], fa_v, sem)
        cpb = pltpu.async_copy(delta_hbm.at[idx_v], fb_v, sem2)
        cpa.wait()
        cpb.wait()

        def row_body(r, _):
            for j in range(HV):
                fa_v[r, pl.ds(j * 16, 16)] = (
                    fa_v[r, pl.ds(j * 16, 16)] + fb_v[r, pl.ds(j * 16, 16)]
                )
            return 0

        lax.fori_loop(0, 16, row_body, 0)
        pltpu.async_copy(fa_v, flat_hbm.at[idx_v], sem).wait()
        return 0

    lax.fori_loop(0, nchunks, body, 0)


_PHASE_B_SCRATCH = [
    pltpu.VMEM((TCHED,), jnp.int32),
    pltpu.VMEM((16,), jnp.int32),
    pltpu.VMEM((16, H), jnp.float32),
    pltpu.VMEM((16, H), jnp.float32),
    pltpu.VMEM((16,), jnp.int32),
    pltpu.SemaphoreType.DMA,
    pltpu.SemaphoreType.DMA,
]


@functools.lru_cache(maxsize=None)
def _make_phase_b():
    return pl.kernel(
        _phase_b_body,
        out_type=(),
        mesh=_mesh(),
        scratch_types=list(_PHASE_B_SCRATCH),
    )


def kernel(hidden_states, W, st_mask, edges_src, edges_tgt, edges_type, edges_pos):
    E = edges_src.shape[0]
    EP = ((E + G - 1) // G) * G

    bucket = jnp.where(
        edges_type == 1, 0,
        jnp.where(edges_type == 6, 1, jnp.where(edges_type == 11, 2, 3)),
    ).astype(jnp.int32)
    key = bucket * KEY_STRIDE + edges_tgt
    _, src_s, tgt_s = lax.sort((key, edges_src, edges_tgt), num_keys=1)
    src_s = jnp.pad(src_s, (0, EP + 16 - E))
    tgt_s = jnp.pad(tgt_s, (0, EP + G - E))

    offs = ((jnp.arange(33, dtype=jnp.int32) * RPW) // 8 * 8).astype(jnp.int32)

    bnd = (jnp.arange(3, dtype=jnp.int32)[:, None] * KEY_STRIDE
           + offs[None, :])
    starts = jnp.sum(key[None, :] < bnd.reshape(-1)[:, None], axis=1,
                     dtype=jnp.int32)
    starts = starts.reshape(3, 33)
    starts = jnp.pad(starts, ((0, 0), (0, 15)))

    flat0 = _build_flat(hidden_states, W, st_mask).reshape(NB, H)
    flat_ref = jax.new_ref(flat0)

    phase_a = _make_phase_a(EP)
    phase_b = _make_phase_b()
    for k in range(3):
        delta, touched, counts = phase_a(flat_ref, src_s, tgt_s, starts[k])
        phase_b(delta, touched, counts, flat_ref)

    return flat_ref[...].reshape(B, N_NODES, H)
